# Initial kernel scaffold; baseline (speedup 1.0000x reference)
import jax
import jax.numpy as jnp
from jax.experimental import pallas as pl


def kernel(x, edge_index, edge_attr, W1, b1, W2, b2, W3, b3, W4, b4):
    raise NotImplementedError("write your pallas kernel here")


# speedup vs baseline: 3.1953x; 3.1953x over previous
import jax
import jax.numpy as jnp
from jax import lax
from jax.experimental import pallas as pl
from jax.experimental.pallas import tpu as pltpu
from jax.experimental.pallas import tpu_sc as plsc

N_NODES = 10000
N_EDGES = 320000
D = 128
DE = 16

NC = 2
NS = 16
NW = NC * NS
EPW = N_EDGES // NW
CH = 80
NCH = EPW // CH
NP = 10240
RPT = NP // NS

EBLK = 2000
NEB = N_EDGES // EBLK
NBLK = 1000
NNB = N_NODES // NBLK


def _proj_body(x_ref, wa_ref, wb_ref, b1_ref, p_ref, q_ref):
    xb = x_ref[...]
    p_ref[...] = jnp.dot(xb, wa_ref[...], preferred_element_type=jnp.float32)
    q_ref[...] = (jnp.dot(xb, wb_ref[...], preferred_element_type=jnp.float32)
                  + b1_ref[...])


def _gather_body(p_hbm, q_hbm, src_hbm, dst_hbm, gs_hbm, gd_hbm,
                 idxs_v, idxd_v, buf_v, sem):
    c = lax.axis_index("c")
    s = lax.axis_index("s")
    wid = s * NC + c
    base = wid * EPW
    pltpu.sync_copy(src_hbm.at[wid], idxs_v)
    pltpu.sync_copy(dst_hbm.at[wid], idxd_v)

    def step(ch, carry):
        off = base + ch * CH
        pltpu.async_copy(p_hbm.at[idxs_v.at[ch]], buf_v, sem).wait()
        pltpu.sync_copy(buf_v, gs_hbm.at[pl.ds(off, CH)])
        pltpu.async_copy(q_hbm.at[idxd_v.at[ch]], buf_v, sem).wait()
        pltpu.sync_copy(buf_v, gd_hbm.at[pl.ds(off, CH)])
        return carry

    lax.fori_loop(0, NCH, step, 0)


def _mlp_body(gs_ref, gd_ref, a_ref, w1c_ref, w2_ref, b2_ref, w3_ref, b3_ref,
              w4_ref, b4_ref, msg_ref):
    h = gs_ref[...] + gd_ref[...] + jnp.dot(
        a_ref[...], w1c_ref[...], preferred_element_type=jnp.float32)
    h = jnp.maximum(h, 0.0)
    h = jnp.maximum(
        jnp.dot(h, w2_ref[...], preferred_element_type=jnp.float32)
        + b2_ref[...], 0.0)
    h = jnp.maximum(
        jnp.dot(h, w3_ref[...], preferred_element_type=jnp.float32)
        + b3_ref[...], 0.0)
    msg_ref[...] = jnp.dot(
        h, w4_ref[...], preferred_element_type=jnp.float32) + b4_ref[...]


def _scatter_body(msg_hbm, dst_hbm, zero_hbm, part_hbm, idx_v, mbuf_v, acc_sh):
    c = lax.axis_index("c")
    s = lax.axis_index("s")
    wid = s * NC + c
    row0 = s * RPT
    pltpu.sync_copy(zero_hbm.at[pl.ds(row0, RPT)], acc_sh.at[pl.ds(row0, RPT)])
    pltpu.sync_copy(dst_hbm.at[wid], idx_v)
    plsc.subcore_barrier()
    base = wid * EPW

    def step(ch, carry):
        pltpu.sync_copy(msg_hbm.at[pl.ds(base + ch * CH, CH)], mbuf_v)
        pltpu.sync_copy(mbuf_v, acc_sh.at[idx_v.at[ch]], add=True)
        return carry

    lax.fori_loop(0, NCH, step, 0)
    plsc.subcore_barrier()
    pltpu.sync_copy(acc_sh.at[pl.ds(row0, RPT)], part_hbm.at[c, pl.ds(row0, RPT)])


def _sum_body(p_ref, o_ref):
    o_ref[...] = p_ref[0] + p_ref[1]


def kernel(x, edge_index, edge_attr, W1, b1, W2, b2, W3, b3, W4, b4):
    W1a = W1[:D]
    W1b = W1[D:2 * D]
    W1c = W1[2 * D:]
    src3 = edge_index[0].reshape(NW, NCH, CH)
    dst3 = edge_index[1].reshape(NW, NCH, CH)
    zeros = jnp.zeros((NP, D), jnp.float32)

    P, Q = pl.pallas_call(
        _proj_body,
        grid=(NNB,),
        in_specs=[pl.BlockSpec((NBLK, D), lambda i: (i, 0)),
                  pl.BlockSpec((D, D), lambda i: (0, 0)),
                  pl.BlockSpec((D, D), lambda i: (0, 0)),
                  pl.BlockSpec((1, D), lambda i: (0, 0))],
        out_specs=[pl.BlockSpec((NBLK, D), lambda i: (i, 0)),
                   pl.BlockSpec((NBLK, D), lambda i: (i, 0))],
        out_shape=[jax.ShapeDtypeStruct((N_NODES, D), jnp.float32)] * 2,
    )(x, W1a, W1b, b1.reshape(1, D))

    mesh = plsc.VectorSubcoreMesh(core_axis_name="c", subcore_axis_name="s")
    Gs, Gd = pl.kernel(
        _gather_body,
        out_type=[jax.ShapeDtypeStruct((N_EDGES, D), jnp.float32)] * 2,
        mesh=mesh,
        scratch_types=[pltpu.VMEM((NCH, CH), jnp.int32),
                       pltpu.VMEM((NCH, CH), jnp.int32),
                       pltpu.VMEM((CH, D), jnp.float32),
                       pltpu.SemaphoreType.DMA],
    )(P, Q, src3, dst3)

    messages = pl.pallas_call(
        _mlp_body,
        grid=(NEB,),
        in_specs=[pl.BlockSpec((EBLK, D), lambda i: (i, 0)),
                  pl.BlockSpec((EBLK, D), lambda i: (i, 0)),
                  pl.BlockSpec((EBLK, DE), lambda i: (i, 0)),
                  pl.BlockSpec((DE, D), lambda i: (0, 0)),
                  pl.BlockSpec((D, D), lambda i: (0, 0)),
                  pl.BlockSpec((1, D), lambda i: (0, 0)),
                  pl.BlockSpec((D, D), lambda i: (0, 0)),
                  pl.BlockSpec((1, D), lambda i: (0, 0)),
                  pl.BlockSpec((D, D), lambda i: (0, 0)),
                  pl.BlockSpec((1, D), lambda i: (0, 0))],
        out_specs=pl.BlockSpec((EBLK, D), lambda i: (i, 0)),
        out_shape=jax.ShapeDtypeStruct((N_EDGES, D), jnp.float32),
    )(Gs, Gd, edge_attr, W1c, W2, b2.reshape(1, D), W3, b3.reshape(1, D),
      W4, b4.reshape(1, D))

    partials = pl.kernel(
        _scatter_body,
        out_type=jax.ShapeDtypeStruct((NC, NP, D), jnp.float32),
        mesh=mesh,
        scratch_types=[pltpu.VMEM((NCH, CH), jnp.int32),
                       pltpu.VMEM((CH, D), jnp.float32),
                       pltpu.VMEM_SHARED((NP, D), jnp.float32)],
    )(messages, dst3, zeros)

    updates = pl.pallas_call(
        _sum_body,
        grid=(NNB,),
        in_specs=[pl.BlockSpec((NC, NBLK, D), lambda i: (0, i, 0))],
        out_specs=pl.BlockSpec((NBLK, D), lambda i: (i, 0)),
        out_shape=jax.ShapeDtypeStruct((N_NODES, D), jnp.float32),
    )(partials)

    return updates, messages
